# Initial kernel scaffold; baseline (speedup 1.0000x reference)
import jax
import jax.numpy as jnp
from jax.experimental import pallas as pl


def kernel(xyz, points, W1, b1, g1, be1, W2, b2, g2, be2, W3, b3, g3, be3):
    raise NotImplementedError("write your pallas kernel here")


# speedup vs baseline: 13.6008x; 13.6008x over previous
import functools

import jax
import jax.numpy as jnp
from jax import lax
from jax.experimental import pallas as pl
from jax.experimental.pallas import tpu as pltpu
from jax.experimental.pallas import tpu_sc as plsc

B = 8
N = 4096
S = 1024
K = 32
RADIUS = 0.2
C1 = 64
C3 = 128
SBLK = 128
NROW = B * S * K
RBLK = SBLK * K
NG = (B * S) // SBLK

_MAXI = 0x7FFFFFFF


def _fps_body(x_ref, y_ref, z_ref, cx_ref, cy_ref, cz_ref):
    X = x_ref[...]
    Y = y_ref[...]
    Z = z_ref[...]
    sub = lax.broadcasted_iota(jnp.int32, (B, 32, 128), 1)
    lane = lax.broadcasted_iota(jnp.int32, (B, 32, 128), 2)
    flat = sub * 128 + lane
    osub = lax.broadcasted_iota(jnp.int32, (B, 8, 128), 1)
    olane = lax.broadcasted_iota(jnp.int32, (B, 8, 128), 2)
    oflat = osub * 128 + olane

    def body(i, carry):
        dists, far, cxs, cys, czs = carry
        sel = flat == far
        cx = jnp.sum(jnp.where(sel, X, 0.0), axis=(1, 2), keepdims=True)
        cy = jnp.sum(jnp.where(sel, Y, 0.0), axis=(1, 2), keepdims=True)
        cz = jnp.sum(jnp.where(sel, Z, 0.0), axis=(1, 2), keepdims=True)
        rec = oflat == i
        cxs = jnp.where(rec, cx, cxs)
        cys = jnp.where(rec, cy, cys)
        czs = jnp.where(rec, cz, czs)
        dx = X - cx
        dy = Y - cy
        dz = Z - cz
        d = dx * dx + dy * dy + dz * dz
        dists = jnp.minimum(dists, d)
        m = jnp.max(dists, axis=(1, 2), keepdims=True)
        far = jnp.min(jnp.where(dists == m, flat, jnp.int32(N)),
                      axis=(1, 2), keepdims=True)
        return dists, far, cxs, cys, czs

    dists0 = jnp.full((B, 32, 128), 1e10, jnp.float32)
    far0 = jnp.zeros((B, 1, 1), jnp.int32)
    z8 = jnp.zeros((B, 8, 128), jnp.float32)
    _, _, cxs, cys, czs = lax.fori_loop(0, S, body,
                                        (dists0, far0, z8, z8, z8))
    cx_ref[...] = cxs
    cy_ref[...] = cys
    cz_ref[...] = czs


def _fps(xc, yc, zc):
    out = jax.ShapeDtypeStruct((B, 8, 128), jnp.float32)
    return pl.pallas_call(
        _fps_body,
        out_shape=(out, out, out),
    )(xc, yc, zc)


def _ballq_body(c3_ref, p3_ref, a_ref, b_ref, idx_ref):
    cb = c3_ref[0].astype(jnp.bfloat16)
    pb = p3_ref[0].astype(jnp.bfloat16)
    e = jnp.dot(cb, pb, preferred_element_type=jnp.float32)
    sq = (a_ref[0] + b_ref[0]) - 2.0 * e
    d = jnp.sqrt(jnp.maximum(sq, 0.0))
    j = lax.broadcasted_iota(jnp.int32, (SBLK, N), 1)
    inf = jnp.float32(jnp.inf)
    nbig = jnp.int32(N)

    cols = []
    first = None
    for k in range(K):
        m = jnp.min(d, axis=1, keepdims=True)
        idx = jnp.min(jnp.where(d == m, j, nbig), axis=1, keepdims=True)
        if k == 0:
            first = idx
            cols.append(idx)
        else:
            cols.append(jnp.where(m > jnp.float32(RADIUS), first, idx))
        d = jnp.where(j == idx, inf, d)
    idx_ref[0] = jnp.concatenate(cols, axis=1)


def _ballq(c3, p3t, asq, bsq):
    return pl.pallas_call(
        _ballq_body,
        grid=(B, S // SBLK),
        in_specs=[
            pl.BlockSpec((1, SBLK, 3), lambda b, i: (b, i, 0)),
            pl.BlockSpec((1, 3, N), lambda b, i: (b, 0, 0)),
            pl.BlockSpec((1, SBLK, 1), lambda b, i: (b, i, 0)),
            pl.BlockSpec((1, 1, N), lambda b, i: (b, 0, 0)),
        ],
        out_specs=pl.BlockSpec((1, SBLK, K), lambda b, i: (b, i, 0)),
        out_shape=jax.ShapeDtypeStruct((B, S, K), jnp.int32),
    )(c3, p3t, asq, bsq)


def _feat_body(p67_ref, nxyz_ref, w_ref, b_ref, f_ref, g_ref):
    w = w_ref[...]
    f_ref[0] = jnp.dot(p67_ref[0], w, preferred_element_type=jnp.float32)
    g_ref[0] = (jnp.dot(nxyz_ref[0], w[:3, :],
                        preferred_element_type=jnp.float32)
                - b_ref[...])


def _feat(p67, nxyz, w1t, b1):
    return pl.pallas_call(
        _feat_body,
        grid=(B,),
        in_specs=[
            pl.BlockSpec((1, N, 67), lambda b: (b, 0, 0)),
            pl.BlockSpec((1, S, 3), lambda b: (b, 0, 0)),
            pl.BlockSpec((67, C1), lambda b: (0, 0)),
            pl.BlockSpec((1, C1), lambda b: (0, 0)),
        ],
        out_specs=[
            pl.BlockSpec((1, N, C1), lambda b: (b, 0, 0)),
            pl.BlockSpec((1, S, C1), lambda b: (b, 0, 0)),
        ],
        out_shape=[
            jax.ShapeDtypeStruct((B, N, C1), jnp.float32),
            jax.ShapeDtypeStruct((B, S, C1), jnp.float32),
        ],
    )(p67, nxyz, w1t, b1)


_CHUNK = 128
_NWORK = 32
_PERW = NROW // _NWORK
_NCH = _PERW // _CHUNK


def _gather_kernel_body(table_hbm, idx_hbm, out_hbm,
                        idx_a, idx_b, rows_a, rows_b, sem_a, sem_b):
    c = lax.axis_index("c")
    s = lax.axis_index("s")
    wid = s * 2 + c
    base = wid * _PERW
    boff = (wid // 4) * N

    def fetch_idx(g, idx_v):
        pltpu.sync_copy(idx_hbm.at[pl.ds(base + g * _CHUNK, _CHUNK)], idx_v)
        for t in range(_CHUNK // 16):
            sl = pl.ds(t * 16, 16)
            idx_v[sl] = idx_v[sl] + boff

    fetch_idx(0, idx_a)
    pltpu.async_copy(table_hbm.at[idx_a], rows_a, sem_a)

    def body(h, _):
        ga = 2 * h
        gb = 2 * h + 1
        fetch_idx(gb, idx_b)
        pltpu.async_copy(table_hbm.at[idx_b], rows_b, sem_b)
        pltpu.make_async_copy(table_hbm.at[idx_a], rows_a, sem_a).wait()
        pltpu.sync_copy(rows_a, out_hbm.at[pl.ds(base + ga * _CHUNK, _CHUNK)])

        @pl.when(h + 1 < _NCH // 2)
        def _():
            fetch_idx(ga + 2, idx_a)
            pltpu.async_copy(table_hbm.at[idx_a], rows_a, sem_a)

        pltpu.make_async_copy(table_hbm.at[idx_b], rows_b, sem_b).wait()
        pltpu.sync_copy(rows_b, out_hbm.at[pl.ds(base + gb * _CHUNK, _CHUNK)])
        return 0

    lax.fori_loop(0, _NCH // 2, body, 0)


def _gather_sc(table, idxf):
    mesh = plsc.VectorSubcoreMesh(core_axis_name="c", subcore_axis_name="s")
    k = pl.kernel(
        _gather_kernel_body,
        out_type=jax.ShapeDtypeStruct((NROW, C1), jnp.float32),
        mesh=mesh,
        compiler_params=pltpu.CompilerParams(use_tc_tiling_on_sc=False),
        scratch_types=[
            pltpu.VMEM((_CHUNK,), jnp.int32),
            pltpu.VMEM((_CHUNK,), jnp.int32),
            pltpu.VMEM((_CHUNK, C1), jnp.float32),
            pltpu.VMEM((_CHUNK, C1), jnp.float32),
            pltpu.SemaphoreType.DMA,
            pltpu.SemaphoreType.DMA,
        ],
    )
    return k(table, idxf)


def _stats1_body(xg_ref, gp_ref, st_ref):
    x3 = xg_ref[...].reshape(SBLK, K, C1)
    g3 = gp_ref[...].reshape(SBLK, 1, C1)
    x1 = x3 - g3
    sm = jnp.sum(x1, axis=(0, 1))[None, :]
    ss = jnp.sum(x1 * x1, axis=(0, 1))[None, :]
    st_ref[0] = jnp.concatenate([sm, ss, jnp.zeros((6, C1), jnp.float32)], 0)


def _stats1(xg, gp):
    return pl.pallas_call(
        _stats1_body,
        grid=(NG,),
        in_specs=[
            pl.BlockSpec((RBLK, C1), lambda i: (i, 0)),
            pl.BlockSpec((SBLK, C1), lambda i: (i, 0)),
        ],
        out_specs=pl.BlockSpec((1, 8, C1), lambda i: (i, 0, 0)),
        out_shape=jax.ShapeDtypeStruct((NG, 8, C1), jnp.float32),
    )(xg, gp)


def _layer2_body(xg_ref, gp_ref, a_ref, c_ref, w_ref, x2_ref, st_ref):
    x3 = xg_ref[...].reshape(SBLK, K, C1)
    g3 = gp_ref[...].reshape(SBLK, 1, C1)
    a = a_ref[...].reshape(1, 1, C1)
    c = c_ref[...].reshape(1, 1, C1)
    h = jnp.maximum((x3 - g3) * a + c, 0.0).reshape(RBLK, C1)
    x2 = jnp.dot(h, w_ref[...], preferred_element_type=jnp.float32)
    x2_ref[...] = x2
    sm = jnp.sum(x2, axis=0)[None, :]
    ss = jnp.sum(x2 * x2, axis=0)[None, :]
    st_ref[0] = jnp.concatenate([sm, ss, jnp.zeros((6, C1), jnp.float32)], 0)


def _layer2(xg, gp, a1, c1, w2t):
    return pl.pallas_call(
        _layer2_body,
        grid=(NG,),
        in_specs=[
            pl.BlockSpec((RBLK, C1), lambda i: (i, 0)),
            pl.BlockSpec((SBLK, C1), lambda i: (i, 0)),
            pl.BlockSpec((1, C1), lambda i: (0, 0)),
            pl.BlockSpec((1, C1), lambda i: (0, 0)),
            pl.BlockSpec((C1, C1), lambda i: (0, 0)),
        ],
        out_specs=[
            pl.BlockSpec((RBLK, C1), lambda i: (i, 0)),
            pl.BlockSpec((1, 8, C1), lambda i: (i, 0, 0)),
        ],
        out_shape=[
            jax.ShapeDtypeStruct((NROW, C1), jnp.float32),
            jax.ShapeDtypeStruct((NG, 8, C1), jnp.float32),
        ],
    )(xg, gp, a1, c1, w2t)


def _layer3_body(x2_ref, a_ref, c_ref, w_ref, mx_ref, mn_ref, st_ref):
    a = a_ref[...]
    c = c_ref[...]
    h = jnp.maximum(x2_ref[...] * a + c, 0.0)
    x3 = jnp.dot(h, w_ref[...], preferred_element_type=jnp.float32)
    sm = jnp.sum(x3, axis=0)[None, :]
    ss = jnp.sum(x3 * x3, axis=0)[None, :]
    st_ref[0] = jnp.concatenate([sm, ss, jnp.zeros((6, C3), jnp.float32)], 0)
    x33 = x3.reshape(SBLK, K, C3)
    mx_ref[...] = jnp.max(x33, axis=1)
    mn_ref[...] = jnp.min(x33, axis=1)


def _layer3(x2, a2, c2, w3t):
    return pl.pallas_call(
        _layer3_body,
        grid=(NG,),
        in_specs=[
            pl.BlockSpec((RBLK, C1), lambda i: (i, 0)),
            pl.BlockSpec((1, C1), lambda i: (0, 0)),
            pl.BlockSpec((1, C1), lambda i: (0, 0)),
            pl.BlockSpec((C1, C3), lambda i: (0, 0)),
        ],
        out_specs=[
            pl.BlockSpec((SBLK, C3), lambda i: (i, 0)),
            pl.BlockSpec((SBLK, C3), lambda i: (i, 0)),
            pl.BlockSpec((1, 8, C3), lambda i: (i, 0, 0)),
        ],
        out_shape=[
            jax.ShapeDtypeStruct((B * S, C3), jnp.float32),
            jax.ShapeDtypeStruct((B * S, C3), jnp.float32),
            jax.ShapeDtypeStruct((NG, 8, C3), jnp.float32),
        ],
    )(x2, a2, c2, w3t)


def _final_body(mx_ref, mn_ref, a_ref, c_ref, o_ref):
    a = a_ref[...]
    c = c_ref[...]
    hi = jnp.maximum(mx_ref[...] * a + c, 0.0)
    lo = jnp.maximum(mn_ref[...] * a + c, 0.0)
    o_ref[...] = jnp.maximum(hi, lo)


def _final(mx, mn, a3, c3):
    return pl.pallas_call(
        _final_body,
        grid=(NG,),
        in_specs=[
            pl.BlockSpec((SBLK, C3), lambda i: (i, 0)),
            pl.BlockSpec((SBLK, C3), lambda i: (i, 0)),
            pl.BlockSpec((1, C3), lambda i: (0, 0)),
            pl.BlockSpec((1, C3), lambda i: (0, 0)),
        ],
        out_specs=pl.BlockSpec((SBLK, C3), lambda i: (i, 0)),
        out_shape=jax.ShapeDtypeStruct((B * S, C3), jnp.float32),
    )(mx, mn, a3, c3)


def _bn_coefs(stats, g, be, ntot):
    tot = jnp.sum(stats, axis=0)
    mean = tot[0] / ntot
    var = tot[1] / ntot - mean * mean
    a = g / jnp.sqrt(var + 1e-5)
    c = be - mean * a
    return a[None, :], c[None, :]


def kernel(xyz, points, W1, b1, g1, be1, W2, b2, g2, be2, W3, b3, g3, be3):
    xc = xyz[:, :, 0].reshape(B, 32, 128)
    yc = xyz[:, :, 1].reshape(B, 32, 128)
    zc = xyz[:, :, 2].reshape(B, 32, 128)

    cxs, cys, czs = _fps(xc, yc, zc)
    cx = cxs.reshape(B, S, 1)
    cy = cys.reshape(B, S, 1)
    cz = czs.reshape(B, S, 1)
    new_xyz = jnp.concatenate([cx, cy, cz], axis=-1)

    asq = jnp.sum(new_xyz ** 2, -1)[:, :, None]
    bsq = jnp.sum(xyz ** 2, -1)[:, None, :]
    idx = _ballq(new_xyz, jnp.swapaxes(xyz, 1, 2), asq, bsq)

    p67 = jnp.concatenate([xyz, points], axis=-1)
    f, gp = _feat(p67, new_xyz, W1.T, b1[None, :])
    table = f.reshape(B * N, C1)
    gpf = gp.reshape(B * S, C1)

    xg = _gather_sc(table, idx.reshape(NROW))

    ntot = jnp.float32(NROW)
    st1 = _stats1(xg, gpf)
    a1, c1 = _bn_coefs(st1, g1, be1, ntot)

    x2, st2 = _layer2(xg, gpf, a1, c1, W2.T)
    a2, c2 = _bn_coefs(st2, g2, be2, ntot)

    mx, mn, st3 = _layer3(x2, a2, c2, W3.T)
    a3, c3 = _bn_coefs(st3, g3, be3, ntot)

    new_points = _final(mx, mn, a3, c3).reshape(B, S, C3)
    return (new_xyz, new_points)


# speedup vs baseline: 16.4875x; 1.2123x over previous
import functools

import jax
import jax.numpy as jnp
from jax import lax
from jax.experimental import pallas as pl
from jax.experimental.pallas import tpu as pltpu
from jax.experimental.pallas import tpu_sc as plsc

B = 8
N = 4096
S = 1024
K = 32
RADIUS = 0.2
C1 = 64
C3 = 128
SBLK = 128
NROW = B * S * K
RBLK = SBLK * K
NG = (B * S) // SBLK

_MAXI = 0x7FFFFFFF


def _fps_body(x_ref, y_ref, z_ref, cx_ref, cy_ref, cz_ref):
    X = x_ref[...]
    Y = y_ref[...]
    Z = z_ref[...]
    sub = lax.broadcasted_iota(jnp.int32, (B, 32, 128), 1)
    lane = lax.broadcasted_iota(jnp.int32, (B, 32, 128), 2)
    flat = sub * 128 + lane
    osub = lax.broadcasted_iota(jnp.int32, (B, 8, 128), 1)
    olane = lax.broadcasted_iota(jnp.int32, (B, 8, 128), 2)
    oflat = osub * 128 + olane

    def body(i, carry):
        dists, far, cxs, cys, czs = carry
        sel = flat == far
        cx = jnp.sum(jnp.where(sel, X, 0.0), axis=(1, 2), keepdims=True)
        cy = jnp.sum(jnp.where(sel, Y, 0.0), axis=(1, 2), keepdims=True)
        cz = jnp.sum(jnp.where(sel, Z, 0.0), axis=(1, 2), keepdims=True)
        rec = oflat == i
        cxs = jnp.where(rec, cx, cxs)
        cys = jnp.where(rec, cy, cys)
        czs = jnp.where(rec, cz, czs)
        dx = X - cx
        dy = Y - cy
        dz = Z - cz
        d = dx * dx + dy * dy + dz * dz
        dists = jnp.minimum(dists, d)
        m = jnp.max(dists, axis=(1, 2), keepdims=True)
        far = jnp.min(jnp.where(dists == m, flat, jnp.int32(N)),
                      axis=(1, 2), keepdims=True)
        return dists, far, cxs, cys, czs

    dists0 = jnp.full((B, 32, 128), 1e10, jnp.float32)
    far0 = jnp.zeros((B, 1, 1), jnp.int32)
    z8 = jnp.zeros((B, 8, 128), jnp.float32)
    _, _, cxs, cys, czs = lax.fori_loop(0, S, body,
                                        (dists0, far0, z8, z8, z8))
    cx_ref[...] = cxs
    cy_ref[...] = cys
    cz_ref[...] = czs


def _fps(xc, yc, zc):
    out = jax.ShapeDtypeStruct((B, 8, 128), jnp.float32)
    return pl.pallas_call(
        _fps_body,
        out_shape=(out, out, out),
    )(xc, yc, zc)


def _ballq_body(c3_ref, p3_ref, a_ref, b_ref, idx_ref):
    cb = c3_ref[0].astype(jnp.bfloat16)
    pb = p3_ref[0].astype(jnp.bfloat16)
    e = jnp.dot(cb, pb, preferred_element_type=jnp.float32)
    sq = (a_ref[0] + b_ref[0]) - 2.0 * e
    d = jnp.sqrt(jnp.maximum(sq, 0.0))
    j = lax.broadcasted_iota(jnp.int32, (SBLK, N), 1)
    uq = (d * jnp.float32((2 ** 19 - 2) / RADIUS)).astype(jnp.int32)
    keys = (uq << 12) | j
    keys = jnp.where(d > jnp.float32(RADIUS), jnp.int32(_MAXI), keys)

    cols = []
    first = None
    for k in range(K):
        m = jnp.min(keys, axis=1, keepdims=True)
        idx = m & jnp.int32(0xFFF)
        if k == 0:
            first = idx
            cols.append(idx)
        else:
            cols.append(jnp.where(m == _MAXI, first, idx))
        keys = jnp.where(keys == m, jnp.int32(_MAXI), keys)
    idx_ref[0] = jnp.concatenate(cols, axis=1)


def _ballq(c3, p3t, asq, bsq):
    return pl.pallas_call(
        _ballq_body,
        grid=(B, S // SBLK),
        in_specs=[
            pl.BlockSpec((1, SBLK, 3), lambda b, i: (b, i, 0)),
            pl.BlockSpec((1, 3, N), lambda b, i: (b, 0, 0)),
            pl.BlockSpec((1, SBLK, 1), lambda b, i: (b, i, 0)),
            pl.BlockSpec((1, 1, N), lambda b, i: (b, 0, 0)),
        ],
        out_specs=pl.BlockSpec((1, SBLK, K), lambda b, i: (b, i, 0)),
        out_shape=jax.ShapeDtypeStruct((B, S, K), jnp.int32),
    )(c3, p3t, asq, bsq)


def _feat_body(p67_ref, nxyz_ref, w_ref, b_ref, f_ref, g_ref):
    w = w_ref[...]
    f_ref[0] = jnp.dot(p67_ref[0], w, preferred_element_type=jnp.float32)
    g_ref[0] = (jnp.dot(nxyz_ref[0], w[:3, :],
                        preferred_element_type=jnp.float32)
                - b_ref[...])


def _feat(p67, nxyz, w1t, b1):
    return pl.pallas_call(
        _feat_body,
        grid=(B,),
        in_specs=[
            pl.BlockSpec((1, N, 67), lambda b: (b, 0, 0)),
            pl.BlockSpec((1, S, 3), lambda b: (b, 0, 0)),
            pl.BlockSpec((67, C1), lambda b: (0, 0)),
            pl.BlockSpec((1, C1), lambda b: (0, 0)),
        ],
        out_specs=[
            pl.BlockSpec((1, N, C1), lambda b: (b, 0, 0)),
            pl.BlockSpec((1, S, C1), lambda b: (b, 0, 0)),
        ],
        out_shape=[
            jax.ShapeDtypeStruct((B, N, C1), jnp.float32),
            jax.ShapeDtypeStruct((B, S, C1), jnp.float32),
        ],
    )(p67, nxyz, w1t, b1)


_CHUNK = 128
_NWORK = 32
_PERW = NROW // _NWORK
_NCH = _PERW // _CHUNK


def _gather_kernel_body(table_hbm, idx_hbm, out_hbm,
                        idx_a, idx_b, rows_a, rows_b, sem_a, sem_b):
    c = lax.axis_index("c")
    s = lax.axis_index("s")
    wid = s * 2 + c
    base = wid * _PERW
    boff = (wid // 4) * N

    def fetch_idx(g, idx_v):
        pltpu.sync_copy(idx_hbm.at[pl.ds(base + g * _CHUNK, _CHUNK)], idx_v)
        for t in range(_CHUNK // 16):
            sl = pl.ds(t * 16, 16)
            idx_v[sl] = idx_v[sl] + boff

    fetch_idx(0, idx_a)
    pltpu.async_copy(table_hbm.at[idx_a], rows_a, sem_a)

    def body(h, _):
        ga = 2 * h
        gb = 2 * h + 1
        fetch_idx(gb, idx_b)
        pltpu.async_copy(table_hbm.at[idx_b], rows_b, sem_b)
        pltpu.make_async_copy(table_hbm.at[idx_a], rows_a, sem_a).wait()
        pltpu.sync_copy(rows_a, out_hbm.at[pl.ds(base + ga * _CHUNK, _CHUNK)])

        @pl.when(h + 1 < _NCH // 2)
        def _():
            fetch_idx(ga + 2, idx_a)
            pltpu.async_copy(table_hbm.at[idx_a], rows_a, sem_a)

        pltpu.make_async_copy(table_hbm.at[idx_b], rows_b, sem_b).wait()
        pltpu.sync_copy(rows_b, out_hbm.at[pl.ds(base + gb * _CHUNK, _CHUNK)])
        return 0

    lax.fori_loop(0, _NCH // 2, body, 0)


def _gather_sc(table, idxf):
    mesh = plsc.VectorSubcoreMesh(core_axis_name="c", subcore_axis_name="s")
    k = pl.kernel(
        _gather_kernel_body,
        out_type=jax.ShapeDtypeStruct((NROW, C1), jnp.float32),
        mesh=mesh,
        compiler_params=pltpu.CompilerParams(use_tc_tiling_on_sc=False),
        scratch_types=[
            pltpu.VMEM((_CHUNK,), jnp.int32),
            pltpu.VMEM((_CHUNK,), jnp.int32),
            pltpu.VMEM((_CHUNK, C1), jnp.float32),
            pltpu.VMEM((_CHUNK, C1), jnp.float32),
            pltpu.SemaphoreType.DMA,
            pltpu.SemaphoreType.DMA,
        ],
    )
    return k(table, idxf)


def _stats1_body(xg_ref, gp_ref, st_ref):
    x3 = xg_ref[...].reshape(SBLK, K, C1)
    g3 = gp_ref[...].reshape(SBLK, 1, C1)
    x1 = x3 - g3
    sm = jnp.sum(x1, axis=(0, 1))[None, :]
    ss = jnp.sum(x1 * x1, axis=(0, 1))[None, :]
    st_ref[0] = jnp.concatenate([sm, ss, jnp.zeros((6, C1), jnp.float32)], 0)


def _stats1(xg, gp):
    return pl.pallas_call(
        _stats1_body,
        grid=(NG,),
        in_specs=[
            pl.BlockSpec((RBLK, C1), lambda i: (i, 0)),
            pl.BlockSpec((SBLK, C1), lambda i: (i, 0)),
        ],
        out_specs=pl.BlockSpec((1, 8, C1), lambda i: (i, 0, 0)),
        out_shape=jax.ShapeDtypeStruct((NG, 8, C1), jnp.float32),
    )(xg, gp)


def _layer2_body(xg_ref, gp_ref, a_ref, c_ref, w_ref, x2_ref, st_ref):
    x3 = xg_ref[...].reshape(SBLK, K, C1)
    g3 = gp_ref[...].reshape(SBLK, 1, C1)
    a = a_ref[...].reshape(1, 1, C1)
    c = c_ref[...].reshape(1, 1, C1)
    h = jnp.maximum((x3 - g3) * a + c, 0.0).reshape(RBLK, C1)
    x2 = jnp.dot(h, w_ref[...], preferred_element_type=jnp.float32)
    x2_ref[...] = x2
    sm = jnp.sum(x2, axis=0)[None, :]
    ss = jnp.sum(x2 * x2, axis=0)[None, :]
    st_ref[0] = jnp.concatenate([sm, ss, jnp.zeros((6, C1), jnp.float32)], 0)


def _layer2(xg, gp, a1, c1, w2t):
    return pl.pallas_call(
        _layer2_body,
        grid=(NG,),
        in_specs=[
            pl.BlockSpec((RBLK, C1), lambda i: (i, 0)),
            pl.BlockSpec((SBLK, C1), lambda i: (i, 0)),
            pl.BlockSpec((1, C1), lambda i: (0, 0)),
            pl.BlockSpec((1, C1), lambda i: (0, 0)),
            pl.BlockSpec((C1, C1), lambda i: (0, 0)),
        ],
        out_specs=[
            pl.BlockSpec((RBLK, C1), lambda i: (i, 0)),
            pl.BlockSpec((1, 8, C1), lambda i: (i, 0, 0)),
        ],
        out_shape=[
            jax.ShapeDtypeStruct((NROW, C1), jnp.float32),
            jax.ShapeDtypeStruct((NG, 8, C1), jnp.float32),
        ],
    )(xg, gp, a1, c1, w2t)


def _layer3_body(x2_ref, a_ref, c_ref, w_ref, mx_ref, mn_ref, st_ref):
    a = a_ref[...]
    c = c_ref[...]
    h = jnp.maximum(x2_ref[...] * a + c, 0.0)
    x3 = jnp.dot(h, w_ref[...], preferred_element_type=jnp.float32)
    sm = jnp.sum(x3, axis=0)[None, :]
    ss = jnp.sum(x3 * x3, axis=0)[None, :]
    st_ref[0] = jnp.concatenate([sm, ss, jnp.zeros((6, C3), jnp.float32)], 0)
    x33 = x3.reshape(SBLK, K, C3)
    mx_ref[...] = jnp.max(x33, axis=1)
    mn_ref[...] = jnp.min(x33, axis=1)


def _layer3(x2, a2, c2, w3t):
    return pl.pallas_call(
        _layer3_body,
        grid=(NG,),
        in_specs=[
            pl.BlockSpec((RBLK, C1), lambda i: (i, 0)),
            pl.BlockSpec((1, C1), lambda i: (0, 0)),
            pl.BlockSpec((1, C1), lambda i: (0, 0)),
            pl.BlockSpec((C1, C3), lambda i: (0, 0)),
        ],
        out_specs=[
            pl.BlockSpec((SBLK, C3), lambda i: (i, 0)),
            pl.BlockSpec((SBLK, C3), lambda i: (i, 0)),
            pl.BlockSpec((1, 8, C3), lambda i: (i, 0, 0)),
        ],
        out_shape=[
            jax.ShapeDtypeStruct((B * S, C3), jnp.float32),
            jax.ShapeDtypeStruct((B * S, C3), jnp.float32),
            jax.ShapeDtypeStruct((NG, 8, C3), jnp.float32),
        ],
    )(x2, a2, c2, w3t)


def _final_body(mx_ref, mn_ref, a_ref, c_ref, o_ref):
    a = a_ref[...]
    c = c_ref[...]
    hi = jnp.maximum(mx_ref[...] * a + c, 0.0)
    lo = jnp.maximum(mn_ref[...] * a + c, 0.0)
    o_ref[...] = jnp.maximum(hi, lo)


def _final(mx, mn, a3, c3):
    return pl.pallas_call(
        _final_body,
        grid=(NG,),
        in_specs=[
            pl.BlockSpec((SBLK, C3), lambda i: (i, 0)),
            pl.BlockSpec((SBLK, C3), lambda i: (i, 0)),
            pl.BlockSpec((1, C3), lambda i: (0, 0)),
            pl.BlockSpec((1, C3), lambda i: (0, 0)),
        ],
        out_specs=pl.BlockSpec((SBLK, C3), lambda i: (i, 0)),
        out_shape=jax.ShapeDtypeStruct((B * S, C3), jnp.float32),
    )(mx, mn, a3, c3)


def _bn_coefs(stats, g, be, ntot):
    tot = jnp.sum(stats, axis=0)
    mean = tot[0] / ntot
    var = tot[1] / ntot - mean * mean
    a = g / jnp.sqrt(var + 1e-5)
    c = be - mean * a
    return a[None, :], c[None, :]


def kernel(xyz, points, W1, b1, g1, be1, W2, b2, g2, be2, W3, b3, g3, be3):
    xc = xyz[:, :, 0].reshape(B, 32, 128)
    yc = xyz[:, :, 1].reshape(B, 32, 128)
    zc = xyz[:, :, 2].reshape(B, 32, 128)

    cxs, cys, czs = _fps(xc, yc, zc)
    cx = cxs.reshape(B, S, 1)
    cy = cys.reshape(B, S, 1)
    cz = czs.reshape(B, S, 1)
    new_xyz = jnp.concatenate([cx, cy, cz], axis=-1)

    asq = jnp.sum(new_xyz ** 2, -1)[:, :, None]
    bsq = jnp.sum(xyz ** 2, -1)[:, None, :]
    idx = _ballq(new_xyz, jnp.swapaxes(xyz, 1, 2), asq, bsq)

    p67 = jnp.concatenate([xyz, points], axis=-1)
    f, gp = _feat(p67, new_xyz, W1.T, b1[None, :])
    table = f.reshape(B * N, C1)
    gpf = gp.reshape(B * S, C1)

    xg = _gather_sc(table, idx.reshape(NROW))

    ntot = jnp.float32(NROW)
    st1 = _stats1(xg, gpf)
    a1, c1 = _bn_coefs(st1, g1, be1, ntot)

    x2, st2 = _layer2(xg, gpf, a1, c1, W2.T)
    a2, c2 = _bn_coefs(st2, g2, be2, ntot)

    mx, mn, st3 = _layer3(x2, a2, c2, W3.T)
    a3, c3 = _bn_coefs(st3, g3, be3, ntot)

    new_points = _final(mx, mn, a3, c3).reshape(B, S, C3)
    return (new_xyz, new_points)


# speedup vs baseline: 17.4888x; 1.0607x over previous
import functools

import jax
import jax.numpy as jnp
from jax import lax
from jax.experimental import pallas as pl
from jax.experimental.pallas import tpu as pltpu
from jax.experimental.pallas import tpu_sc as plsc

B = 8
N = 4096
S = 1024
K = 32
RADIUS = 0.2
C1 = 64
C3 = 128
SBLK = 128
NROW = B * S * K
RBLK = SBLK * K
NG = (B * S) // SBLK

_MAXI = 0x7FFFFFFF


def _fps_body(x_ref, y_ref, z_ref, cx_ref, cy_ref, cz_ref):
    X = x_ref[...]
    Y = y_ref[...]
    Z = z_ref[...]
    sub = lax.broadcasted_iota(jnp.int32, (B, 32, 128), 1)
    lane = lax.broadcasted_iota(jnp.int32, (B, 32, 128), 2)
    flat = sub * 128 + lane
    osub = lax.broadcasted_iota(jnp.int32, (B, 8, 128), 1)
    olane = lax.broadcasted_iota(jnp.int32, (B, 8, 128), 2)
    oflat = osub * 128 + olane

    def body(i, carry):
        dists, far, cxs, cys, czs = carry
        sel = flat == far
        cx = jnp.sum(jnp.where(sel, X, 0.0), axis=(1, 2), keepdims=True)
        cy = jnp.sum(jnp.where(sel, Y, 0.0), axis=(1, 2), keepdims=True)
        cz = jnp.sum(jnp.where(sel, Z, 0.0), axis=(1, 2), keepdims=True)
        rec = oflat == i
        cxs = jnp.where(rec, cx, cxs)
        cys = jnp.where(rec, cy, cys)
        czs = jnp.where(rec, cz, czs)
        dx = X - cx
        dy = Y - cy
        dz = Z - cz
        d = dx * dx + dy * dy + dz * dz
        dists = jnp.minimum(dists, d)
        m = jnp.max(dists, axis=(1, 2), keepdims=True)
        far = jnp.min(jnp.where(dists == m, flat, jnp.int32(N)),
                      axis=(1, 2), keepdims=True)
        return dists, far, cxs, cys, czs

    def body2(h, carry):
        return body(2 * h + 1, body(2 * h, carry))

    dists0 = jnp.full((B, 32, 128), 1e10, jnp.float32)
    far0 = jnp.zeros((B, 1, 1), jnp.int32)
    z8 = jnp.zeros((B, 8, 128), jnp.float32)
    _, _, cxs, cys, czs = lax.fori_loop(0, S // 2, body2,
                                        (dists0, far0, z8, z8, z8))
    cx_ref[...] = cxs
    cy_ref[...] = cys
    cz_ref[...] = czs


def _fps(xc, yc, zc):
    out = jax.ShapeDtypeStruct((B, 8, 128), jnp.float32)
    return pl.pallas_call(
        _fps_body,
        out_shape=(out, out, out),
    )(xc, yc, zc)


QBLK = 256


def _ballq_body(c3_ref, p3_ref, a_ref, b_ref, idx_ref):
    cb = c3_ref[0].astype(jnp.bfloat16)
    pb = p3_ref[0].astype(jnp.bfloat16)
    e = jnp.dot(cb, pb, preferred_element_type=jnp.float32)
    sq = (a_ref[0] + b_ref[0]) - 2.0 * e
    d = jnp.sqrt(jnp.maximum(sq, 0.0))
    j = lax.broadcasted_iota(jnp.int32, (QBLK, N), 1)
    uq = (d * jnp.float32((2 ** 19 - 2) / RADIUS)).astype(jnp.int32)
    keys = (uq << 12) | j
    keys = jnp.where(d > jnp.float32(RADIUS), jnp.int32(_MAXI), keys)

    cols = []
    first = None
    for k in range(K):
        m = jnp.min(keys, axis=1, keepdims=True)
        idx = m & jnp.int32(0xFFF)
        if k == 0:
            first = idx
            cols.append(idx)
        else:
            cols.append(jnp.where(m == _MAXI, first, idx))
        keys = jnp.where(keys == m, jnp.int32(_MAXI), keys)
    idx_ref[0] = jnp.concatenate(cols, axis=1)


def _ballq(c3, p3t, asq, bsq):
    return pl.pallas_call(
        _ballq_body,
        grid=(B, S // QBLK),
        in_specs=[
            pl.BlockSpec((1, QBLK, 3), lambda b, i: (b, i, 0)),
            pl.BlockSpec((1, 3, N), lambda b, i: (b, 0, 0)),
            pl.BlockSpec((1, QBLK, 1), lambda b, i: (b, i, 0)),
            pl.BlockSpec((1, 1, N), lambda b, i: (b, 0, 0)),
        ],
        out_specs=pl.BlockSpec((1, QBLK, K), lambda b, i: (b, i, 0)),
        out_shape=jax.ShapeDtypeStruct((B, S, K), jnp.int32),
    )(c3, p3t, asq, bsq)


def _feat_body(p67_ref, nxyz_ref, w_ref, b_ref, f_ref, g_ref):
    w = w_ref[...]
    f_ref[0] = jnp.dot(p67_ref[0], w, preferred_element_type=jnp.float32)
    g_ref[0] = (jnp.dot(nxyz_ref[0], w[:3, :],
                        preferred_element_type=jnp.float32)
                - b_ref[...])


def _feat(p67, nxyz, w1t, b1):
    return pl.pallas_call(
        _feat_body,
        grid=(B,),
        in_specs=[
            pl.BlockSpec((1, N, 67), lambda b: (b, 0, 0)),
            pl.BlockSpec((1, S, 3), lambda b: (b, 0, 0)),
            pl.BlockSpec((67, C1), lambda b: (0, 0)),
            pl.BlockSpec((1, C1), lambda b: (0, 0)),
        ],
        out_specs=[
            pl.BlockSpec((1, N, C1), lambda b: (b, 0, 0)),
            pl.BlockSpec((1, S, C1), lambda b: (b, 0, 0)),
        ],
        out_shape=[
            jax.ShapeDtypeStruct((B, N, C1), jnp.float32),
            jax.ShapeDtypeStruct((B, S, C1), jnp.float32),
        ],
    )(p67, nxyz, w1t, b1)


_CHUNK = 128
_NWORK = 32
_PERW = NROW // _NWORK
_NCH = _PERW // _CHUNK


def _gather_kernel_body(table_hbm, idx_hbm, out_hbm,
                        idx_a, idx_b, rows_a, rows_b, sem_a, sem_b):
    c = lax.axis_index("c")
    s = lax.axis_index("s")
    wid = s * 2 + c
    base = wid * _PERW
    boff = (wid // 4) * N

    def fetch_idx(g, idx_v):
        pltpu.sync_copy(idx_hbm.at[pl.ds(base + g * _CHUNK, _CHUNK)], idx_v)
        for t in range(_CHUNK // 16):
            sl = pl.ds(t * 16, 16)
            idx_v[sl] = idx_v[sl] + boff

    fetch_idx(0, idx_a)
    pltpu.async_copy(table_hbm.at[idx_a], rows_a, sem_a)

    def body(h, _):
        ga = 2 * h
        gb = 2 * h + 1
        fetch_idx(gb, idx_b)
        pltpu.async_copy(table_hbm.at[idx_b], rows_b, sem_b)
        pltpu.make_async_copy(table_hbm.at[idx_a], rows_a, sem_a).wait()
        pltpu.sync_copy(rows_a, out_hbm.at[pl.ds(base + ga * _CHUNK, _CHUNK)])

        @pl.when(h + 1 < _NCH // 2)
        def _():
            fetch_idx(ga + 2, idx_a)
            pltpu.async_copy(table_hbm.at[idx_a], rows_a, sem_a)

        pltpu.make_async_copy(table_hbm.at[idx_b], rows_b, sem_b).wait()
        pltpu.sync_copy(rows_b, out_hbm.at[pl.ds(base + gb * _CHUNK, _CHUNK)])
        return 0

    lax.fori_loop(0, _NCH // 2, body, 0)


def _gather_sc(table, idxf):
    mesh = plsc.VectorSubcoreMesh(core_axis_name="c", subcore_axis_name="s")
    k = pl.kernel(
        _gather_kernel_body,
        out_type=jax.ShapeDtypeStruct((NROW, C1), jnp.float32),
        mesh=mesh,
        compiler_params=pltpu.CompilerParams(use_tc_tiling_on_sc=False),
        scratch_types=[
            pltpu.VMEM((_CHUNK,), jnp.int32),
            pltpu.VMEM((_CHUNK,), jnp.int32),
            pltpu.VMEM((_CHUNK, C1), jnp.float32),
            pltpu.VMEM((_CHUNK, C1), jnp.float32),
            pltpu.SemaphoreType.DMA,
            pltpu.SemaphoreType.DMA,
        ],
    )
    return k(table, idxf)


def _stats1_body(xg_ref, gp_ref, st_ref):
    x3 = xg_ref[...].reshape(SBLK, K, C1)
    g3 = gp_ref[...].reshape(SBLK, 1, C1)
    x1 = x3 - g3
    sm = jnp.sum(x1, axis=(0, 1))[None, :]
    ss = jnp.sum(x1 * x1, axis=(0, 1))[None, :]
    st_ref[0] = jnp.concatenate([sm, ss, jnp.zeros((6, C1), jnp.float32)], 0)


def _stats1(xg, gp):
    return pl.pallas_call(
        _stats1_body,
        grid=(NG,),
        in_specs=[
            pl.BlockSpec((RBLK, C1), lambda i: (i, 0)),
            pl.BlockSpec((SBLK, C1), lambda i: (i, 0)),
        ],
        out_specs=pl.BlockSpec((1, 8, C1), lambda i: (i, 0, 0)),
        out_shape=jax.ShapeDtypeStruct((NG, 8, C1), jnp.float32),
    )(xg, gp)


def _layer2_body(xg_ref, gp_ref, a_ref, c_ref, w_ref, x2_ref, st_ref):
    x3 = xg_ref[...].reshape(SBLK, K, C1)
    g3 = gp_ref[...].reshape(SBLK, 1, C1)
    a = a_ref[...].reshape(1, 1, C1)
    c = c_ref[...].reshape(1, 1, C1)
    h = jnp.maximum((x3 - g3) * a + c, 0.0).reshape(RBLK, C1)
    x2 = jnp.dot(h, w_ref[...], preferred_element_type=jnp.float32)
    x2_ref[...] = x2
    sm = jnp.sum(x2, axis=0)[None, :]
    ss = jnp.sum(x2 * x2, axis=0)[None, :]
    st_ref[0] = jnp.concatenate([sm, ss, jnp.zeros((6, C1), jnp.float32)], 0)


def _layer2(xg, gp, a1, c1, w2t):
    return pl.pallas_call(
        _layer2_body,
        grid=(NG,),
        in_specs=[
            pl.BlockSpec((RBLK, C1), lambda i: (i, 0)),
            pl.BlockSpec((SBLK, C1), lambda i: (i, 0)),
            pl.BlockSpec((1, C1), lambda i: (0, 0)),
            pl.BlockSpec((1, C1), lambda i: (0, 0)),
            pl.BlockSpec((C1, C1), lambda i: (0, 0)),
        ],
        out_specs=[
            pl.BlockSpec((RBLK, C1), lambda i: (i, 0)),
            pl.BlockSpec((1, 8, C1), lambda i: (i, 0, 0)),
        ],
        out_shape=[
            jax.ShapeDtypeStruct((NROW, C1), jnp.float32),
            jax.ShapeDtypeStruct((NG, 8, C1), jnp.float32),
        ],
    )(xg, gp, a1, c1, w2t)


def _layer3_body(x2_ref, a_ref, c_ref, w_ref, mx_ref, mn_ref, st_ref):
    a = a_ref[...]
    c = c_ref[...]
    h = jnp.maximum(x2_ref[...] * a + c, 0.0)
    x3 = jnp.dot(h, w_ref[...], preferred_element_type=jnp.float32)
    sm = jnp.sum(x3, axis=0)[None, :]
    ss = jnp.sum(x3 * x3, axis=0)[None, :]
    st_ref[0] = jnp.concatenate([sm, ss, jnp.zeros((6, C3), jnp.float32)], 0)
    x33 = x3.reshape(SBLK, K, C3)
    mx_ref[...] = jnp.max(x33, axis=1)
    mn_ref[...] = jnp.min(x33, axis=1)


def _layer3(x2, a2, c2, w3t):
    return pl.pallas_call(
        _layer3_body,
        grid=(NG,),
        in_specs=[
            pl.BlockSpec((RBLK, C1), lambda i: (i, 0)),
            pl.BlockSpec((1, C1), lambda i: (0, 0)),
            pl.BlockSpec((1, C1), lambda i: (0, 0)),
            pl.BlockSpec((C1, C3), lambda i: (0, 0)),
        ],
        out_specs=[
            pl.BlockSpec((SBLK, C3), lambda i: (i, 0)),
            pl.BlockSpec((SBLK, C3), lambda i: (i, 0)),
            pl.BlockSpec((1, 8, C3), lambda i: (i, 0, 0)),
        ],
        out_shape=[
            jax.ShapeDtypeStruct((B * S, C3), jnp.float32),
            jax.ShapeDtypeStruct((B * S, C3), jnp.float32),
            jax.ShapeDtypeStruct((NG, 8, C3), jnp.float32),
        ],
    )(x2, a2, c2, w3t)


def _final_body(mx_ref, mn_ref, a_ref, c_ref, o_ref):
    a = a_ref[...]
    c = c_ref[...]
    hi = jnp.maximum(mx_ref[...] * a + c, 0.0)
    lo = jnp.maximum(mn_ref[...] * a + c, 0.0)
    o_ref[...] = jnp.maximum(hi, lo)


def _final(mx, mn, a3, c3):
    return pl.pallas_call(
        _final_body,
        grid=(NG,),
        in_specs=[
            pl.BlockSpec((SBLK, C3), lambda i: (i, 0)),
            pl.BlockSpec((SBLK, C3), lambda i: (i, 0)),
            pl.BlockSpec((1, C3), lambda i: (0, 0)),
            pl.BlockSpec((1, C3), lambda i: (0, 0)),
        ],
        out_specs=pl.BlockSpec((SBLK, C3), lambda i: (i, 0)),
        out_shape=jax.ShapeDtypeStruct((B * S, C3), jnp.float32),
    )(mx, mn, a3, c3)


def _bn_coefs(stats, g, be, ntot):
    tot = jnp.sum(stats, axis=0)
    mean = tot[0] / ntot
    var = tot[1] / ntot - mean * mean
    a = g / jnp.sqrt(var + 1e-5)
    c = be - mean * a
    return a[None, :], c[None, :]


def kernel(xyz, points, W1, b1, g1, be1, W2, b2, g2, be2, W3, b3, g3, be3):
    xc = xyz[:, :, 0].reshape(B, 32, 128)
    yc = xyz[:, :, 1].reshape(B, 32, 128)
    zc = xyz[:, :, 2].reshape(B, 32, 128)

    cxs, cys, czs = _fps(xc, yc, zc)
    cx = cxs.reshape(B, S, 1)
    cy = cys.reshape(B, S, 1)
    cz = czs.reshape(B, S, 1)
    new_xyz = jnp.concatenate([cx, cy, cz], axis=-1)

    asq = jnp.sum(new_xyz ** 2, -1)[:, :, None]
    bsq = jnp.sum(xyz ** 2, -1)[:, None, :]
    idx = _ballq(new_xyz, jnp.swapaxes(xyz, 1, 2), asq, bsq)

    p67 = jnp.concatenate([xyz, points], axis=-1)
    f, gp = _feat(p67, new_xyz, W1.T, b1[None, :])
    table = f.reshape(B * N, C1)
    gpf = gp.reshape(B * S, C1)

    xg = _gather_sc(table, idx.reshape(NROW))

    ntot = jnp.float32(NROW)
    st1 = _stats1(xg, gpf)
    a1, c1 = _bn_coefs(st1, g1, be1, ntot)

    x2, st2 = _layer2(xg, gpf, a1, c1, W2.T)
    a2, c2 = _bn_coefs(st2, g2, be2, ntot)

    mx, mn, st3 = _layer3(x2, a2, c2, W3.T)
    a3, c3 = _bn_coefs(st3, g3, be3, ntot)

    new_points = _final(mx, mn, a3, c3).reshape(B, S, C3)
    return (new_xyz, new_points)
